# baseline (speedup 1.0000x reference)
import functools

import jax
import jax.numpy as jnp
from jax import lax
from jax.experimental import pallas as pl
from jax.experimental.pallas import tpu as pltpu
from jax.experimental.pallas import tpu_sc as plsc

B = 16384
D = 128
NC = 2
NS = 16
NW = NC * NS
BPW = B // NW
CH = 128
NCHUNK = BPW // CH

_mesh = plsc.VectorSubcoreMesh(core_axis_name="c", subcore_axis_name="s")


@functools.partial(
    pl.kernel,
    mesh=_mesh,
    compiler_params=pltpu.CompilerParams(needs_layout_passes=False),
    out_type=jax.ShapeDtypeStruct((B,), jnp.float32),
    scratch_types=[
        pltpu.VMEM((CH,), jnp.int32),
        pltpu.VMEM((CH, D), jnp.float32),
        pltpu.VMEM((CH, D), jnp.float32),
        pltpu.VMEM((CH,), jnp.float32),
        pltpu.SemaphoreType.DMA,
    ],
)
def _sim_kernel(x_hbm, idx_hbm, out_hbm, idx_v, own_v, gth_v, out_v, sem):
    wid = lax.axis_index("s") * NC + lax.axis_index("c")
    lanes = lax.iota(jnp.int32, 16)

    def chunk_body(ci, carry):
        base = wid * BPW + ci * CH
        pltpu.sync_copy(idx_hbm.at[pl.ds(base, CH)], idx_v)
        pltpu.async_copy(x_hbm.at[idx_v], gth_v, sem).wait()
        pltpu.sync_copy(x_hbm.at[pl.ds(base, CH)], own_v)

        def row_body(r, carry2):
            acc = jnp.zeros((16,), jnp.float32)
            for k in range(D // 16):
                a = own_v[r, pl.ds(k * 16, 16)]
                b = gth_v[r, pl.ds(k * 16, 16)]
                acc = acc + jnp.abs(a - b)
            s = jnp.sum(acc)
            plsc.store_scatter(
                out_v,
                [jnp.full((16,), r, jnp.int32)],
                jnp.full((16,), -s, jnp.float32),
                mask=lanes == 0,
            )
            return carry2

        lax.fori_loop(0, CH, row_body, 0)

        def exp_body(rb, carry3):
            v = out_v[pl.ds(rb * 16, 16)]
            out_v[pl.ds(rb * 16, 16)] = jnp.exp(v)
            return carry3

        lax.fori_loop(0, CH // 16, exp_body, 0)
        pltpu.sync_copy(out_v, out_hbm.at[pl.ds(base, CH)])
        return carry

    lax.fori_loop(0, NCHUNK, chunk_body, 0)


def kernel(x, idx):
    return _sim_kernel(x, idx).reshape(B, 1)


# speedup vs baseline: 1.2388x; 1.2388x over previous
import functools

import jax
import jax.numpy as jnp
from jax import lax
from jax.experimental import pallas as pl
from jax.experimental.pallas import tpu as pltpu
from jax.experimental.pallas import tpu_sc as plsc

B = 16384
D = 128
NC = 2
NS = 16
NW = NC * NS
BPW = B // NW
CH = 128
NCHUNK = BPW // CH
RUNROLL = 4

_mesh = plsc.VectorSubcoreMesh(core_axis_name="c", subcore_axis_name="s")


@functools.partial(
    pl.kernel,
    mesh=_mesh,
    compiler_params=pltpu.CompilerParams(needs_layout_passes=False),
    out_type=jax.ShapeDtypeStruct((B,), jnp.float32),
    scratch_types=[
        [pltpu.VMEM((CH,), jnp.int32) for _ in range(2)],
        [pltpu.VMEM((CH, D), jnp.float32) for _ in range(2)],
        [pltpu.VMEM((CH, D), jnp.float32) for _ in range(2)],
        [pltpu.VMEM((CH,), jnp.float32) for _ in range(2)],
        [pltpu.SemaphoreType.DMA for _ in range(2)],
        [pltpu.SemaphoreType.DMA for _ in range(2)],
    ],
)
def _sim_kernel(x_hbm, idx_hbm, out_hbm, idx_v, own_v, gth_v, out_v, gsem, osem):
    wid = lax.axis_index("s") * NC + lax.axis_index("c")
    lanes = lax.iota(jnp.int32, 16)
    last_lane = lanes == 15

    def issue(ci, bi):
        base = wid * BPW + ci * CH
        pltpu.sync_copy(idx_hbm.at[pl.ds(base, CH)], idx_v[bi])
        g = pltpu.async_copy(x_hbm.at[idx_v[bi]], gth_v[bi], gsem[bi])
        o = pltpu.async_copy(x_hbm.at[pl.ds(base, CH)], own_v[bi], osem[bi])
        return g, o

    def compute_row(r, ov, gv, outv):
        acc0 = jnp.abs(ov[r, pl.ds(0, 16)] - gv[r, pl.ds(0, 16)])
        acc1 = jnp.abs(ov[r, pl.ds(16, 16)] - gv[r, pl.ds(16, 16)])
        for k in range(2, D // 16, 2):
            acc0 = acc0 + jnp.abs(ov[r, pl.ds(k * 16, 16)] - gv[r, pl.ds(k * 16, 16)])
            acc1 = acc1 + jnp.abs(ov[r, pl.ds(k * 16 + 16, 16)] - gv[r, pl.ds(k * 16 + 16, 16)])
        cs = plsc.cumsum(acc0 + acc1)
        plsc.store_scatter(
            outv, [jnp.full((16,), r, jnp.int32)], -cs, mask=last_lane
        )

    handles = issue(0, 0)
    for ci in range(NCHUNK):
        bi = ci % 2
        nxt = None
        if ci + 1 < NCHUNK:
            nxt = issue(ci + 1, 1 - bi)
        handles[0].wait()
        handles[1].wait()

        ov, gv, outv = own_v[bi], gth_v[bi], out_v[bi]

        def row_group(g, carry):
            for u in range(RUNROLL):
                compute_row(g * RUNROLL + u, ov, gv, outv)
            return carry

        lax.fori_loop(0, CH // RUNROLL, row_group, 0)

        for rb in range(CH // 16):
            outv[pl.ds(rb * 16, 16)] = jnp.exp(outv[pl.ds(rb * 16, 16)])

        base = wid * BPW + ci * CH
        pltpu.sync_copy(outv, out_hbm.at[pl.ds(base, CH)])
        handles = nxt


def kernel(x, idx):
    return _sim_kernel(x, idx).reshape(B, 1)


# speedup vs baseline: 1.5312x; 1.2360x over previous
import functools

import jax
import jax.numpy as jnp
from jax import lax
from jax.experimental import pallas as pl
from jax.experimental.pallas import tpu as pltpu
from jax.experimental.pallas import tpu_sc as plsc

B = 16384
D = 128
NC = 2
NS = 16
NW = NC * NS
BPW = B // NW
CH = 128
NCHUNK = BPW // CH
RUNROLL = 4

_mesh = plsc.VectorSubcoreMesh(core_axis_name="c", subcore_axis_name="s")


@functools.partial(
    pl.kernel,
    mesh=_mesh,
    compiler_params=pltpu.CompilerParams(needs_layout_passes=False),
    out_type=jax.ShapeDtypeStruct((B,), jnp.float32),
    scratch_types=[
        [pltpu.VMEM((CH,), jnp.int32) for _ in range(2)],
        [pltpu.VMEM((CH, D), jnp.float32) for _ in range(2)],
        [pltpu.VMEM((CH, D), jnp.float32) for _ in range(2)],
        [pltpu.VMEM((CH,), jnp.float32) for _ in range(2)],
        [pltpu.SemaphoreType.DMA for _ in range(2)],
        [pltpu.SemaphoreType.DMA for _ in range(2)],
    ],
)
def _sim_kernel(x_hbm, idx_hbm, out_hbm, idx_v, own_v, gth_v, out_v, gsem, osem):
    wid = lax.axis_index("s") * NC + lax.axis_index("c")
    lanes = lax.iota(jnp.int32, 16)
    last_lane = lanes == 15

    def issue(ci, bi):
        base = wid * BPW + ci * CH
        pltpu.sync_copy(idx_hbm.at[pl.ds(base, CH)], idx_v[bi])
        g = pltpu.async_copy(x_hbm.at[idx_v[bi]], gth_v[bi], gsem[bi])
        o = pltpu.async_copy(x_hbm.at[pl.ds(base, CH)], own_v[bi], osem[bi])
        return g, o

    def compute_row(r, ov, gv, outv):
        acc0 = jnp.abs(ov[r, pl.ds(0, 16)] - gv[r, pl.ds(0, 16)])
        acc1 = jnp.abs(ov[r, pl.ds(16, 16)] - gv[r, pl.ds(16, 16)])
        for k in range(2, D // 16, 2):
            acc0 = acc0 + jnp.abs(ov[r, pl.ds(k * 16, 16)] - gv[r, pl.ds(k * 16, 16)])
            acc1 = acc1 + jnp.abs(ov[r, pl.ds(k * 16 + 16, 16)] - gv[r, pl.ds(k * 16 + 16, 16)])
        cs = plsc.cumsum(acc0 + acc1)
        plsc.store_scatter(
            outv, [jnp.full((16,), r, jnp.int32)], -cs, mask=last_lane
        )

    handles = issue(0, 0)
    for ci in range(NCHUNK):
        bi = ci % 2
        nxt = None
        if ci + 1 < NCHUNK:
            nxt = issue(ci + 1, 1 - bi)
        handles[0].wait()
        handles[1].wait()

        ov, gv, outv = own_v[bi], gth_v[bi], out_v[bi]

        @plsc.parallel_loop(0, CH, step=1, unroll=RUNROLL)
        def _rows(r):
            compute_row(r, ov, gv, outv)

        for rb in range(CH // 16):
            outv[pl.ds(rb * 16, 16)] = jnp.exp(outv[pl.ds(rb * 16, 16)])

        base = wid * BPW + ci * CH
        pltpu.sync_copy(outv, out_hbm.at[pl.ds(base, CH)])
        handles = nxt


def kernel(x, idx):
    return _sim_kernel(x, idx).reshape(B, 1)


# speedup vs baseline: 1.5707x; 1.0258x over previous
import functools

import jax
import jax.numpy as jnp
from jax import lax
from jax.experimental import pallas as pl
from jax.experimental.pallas import tpu as pltpu
from jax.experimental.pallas import tpu_sc as plsc

B = 16384
D = 128
NC = 2
NS = 16
NW = NC * NS
BPW = B // NW
CH = 128
NCHUNK = BPW // CH
RUNROLL = 4

_mesh = plsc.VectorSubcoreMesh(core_axis_name="c", subcore_axis_name="s")


@functools.partial(
    pl.kernel,
    mesh=_mesh,
    compiler_params=pltpu.CompilerParams(needs_layout_passes=False),
    out_type=jax.ShapeDtypeStruct((B,), jnp.float32),
    scratch_types=[
        pltpu.VMEM((BPW,), jnp.int32),
        [pltpu.VMEM((CH, D), jnp.float32) for _ in range(2)],
        [pltpu.VMEM((CH, D), jnp.float32) for _ in range(2)],
        pltpu.VMEM((BPW,), jnp.float32),
        [pltpu.SemaphoreType.DMA for _ in range(2)],
        [pltpu.SemaphoreType.DMA for _ in range(2)],
        pltpu.SemaphoreType.DMA,
    ],
)
def _sim_kernel(x_hbm, idx_hbm, out_hbm, idx_v, own_v, gth_v, out_v, gsem, osem, isem):
    wid = lax.axis_index("s") * NC + lax.axis_index("c")
    stripe = wid * BPW
    lanes = lax.iota(jnp.int32, 16)
    last_lane = lanes == 15

    pltpu.async_copy(idx_hbm.at[pl.ds(stripe, BPW)], idx_v, isem).wait()

    def issue(ci, bi):
        g = pltpu.async_copy(
            x_hbm.at[idx_v.at[pl.ds(ci * CH, CH)]], gth_v[bi], gsem[bi]
        )
        o = pltpu.async_copy(
            x_hbm.at[pl.ds(stripe + ci * CH, CH)], own_v[bi], osem[bi]
        )
        return g, o

    def compute_row(r, r_out, ov, gv):
        acc0 = jnp.abs(ov[r, pl.ds(0, 16)] - gv[r, pl.ds(0, 16)])
        acc1 = jnp.abs(ov[r, pl.ds(16, 16)] - gv[r, pl.ds(16, 16)])
        for k in range(2, D // 16, 2):
            acc0 = acc0 + jnp.abs(ov[r, pl.ds(k * 16, 16)] - gv[r, pl.ds(k * 16, 16)])
            acc1 = acc1 + jnp.abs(ov[r, pl.ds(k * 16 + 16, 16)] - gv[r, pl.ds(k * 16 + 16, 16)])
        cs = plsc.cumsum(acc0 + acc1)
        plsc.store_scatter(
            out_v, [jnp.full((16,), r_out, jnp.int32)], -cs, mask=last_lane
        )

    handles = issue(0, 0)
    for ci in range(NCHUNK):
        bi = ci % 2
        nxt = None
        if ci + 1 < NCHUNK:
            nxt = issue(ci + 1, 1 - bi)
        handles[0].wait()
        handles[1].wait()

        ov, gv = own_v[bi], gth_v[bi]
        off = ci * CH

        @plsc.parallel_loop(0, CH, step=1, unroll=RUNROLL)
        def _rows(r):
            compute_row(r, off + r, ov, gv)

        handles = nxt

    @plsc.parallel_loop(0, BPW // 16, step=1, unroll=4)
    def _exp(rb):
        out_v[pl.ds(rb * 16, 16)] = jnp.exp(out_v[pl.ds(rb * 16, 16)])

    pltpu.sync_copy(out_v, out_hbm.at[pl.ds(stripe, BPW)])


def kernel(x, idx):
    return _sim_kernel(x, idx).reshape(B, 1)


# speedup vs baseline: 1.5762x; 1.0035x over previous
import functools

import jax
import jax.numpy as jnp
from jax import lax
from jax.experimental import pallas as pl
from jax.experimental.pallas import tpu as pltpu
from jax.experimental.pallas import tpu_sc as plsc

B = 16384
D = 128
NC = 2
NS = 16
NW = NC * NS
BPW = B // NW
CH = 128
NCHUNK = BPW // CH
RUNROLL = 4

_mesh = plsc.VectorSubcoreMesh(core_axis_name="c", subcore_axis_name="s")


@functools.partial(
    pl.kernel,
    mesh=_mesh,
    compiler_params=pltpu.CompilerParams(needs_layout_passes=False),
    out_type=jax.ShapeDtypeStruct((B,), jnp.float32),
    scratch_types=[
        pltpu.VMEM((BPW,), jnp.int32),
        [pltpu.VMEM((CH, D), jnp.float32) for _ in range(2)],
        [pltpu.VMEM((CH, D), jnp.float32) for _ in range(2)],
        pltpu.VMEM((BPW,), jnp.float32),
        [pltpu.SemaphoreType.DMA for _ in range(2)],
        [pltpu.SemaphoreType.DMA for _ in range(2)],
        pltpu.SemaphoreType.DMA,
    ],
)
def _sim_kernel(x_hbm, idx_hbm, out_hbm, idx_v, own_v, gth_v, out_v, gsem, osem, isem):
    wid = lax.axis_index("s") * NC + lax.axis_index("c")
    stripe = wid * BPW
    lanes = lax.iota(jnp.int32, 16)
    last_lane = lanes == 15

    pltpu.async_copy(idx_hbm.at[pl.ds(stripe, BPW)], idx_v, isem).wait()

    def compute_row(r, r_out, ov, gv):
        acc0 = jnp.abs(ov[r, pl.ds(0, 16)] - gv[r, pl.ds(0, 16)])
        acc1 = jnp.abs(ov[r, pl.ds(16, 16)] - gv[r, pl.ds(16, 16)])
        for k in range(2, D // 16, 2):
            acc0 = acc0 + jnp.abs(ov[r, pl.ds(k * 16, 16)] - gv[r, pl.ds(k * 16, 16)])
            acc1 = acc1 + jnp.abs(ov[r, pl.ds(k * 16 + 16, 16)] - gv[r, pl.ds(k * 16 + 16, 16)])
        cs = plsc.cumsum(acc0 + acc1)
        plsc.store_scatter(
            out_v, [jnp.full((16,), r_out, jnp.int32)], -cs, mask=last_lane
        )

    W0 = 32
    pieces = [(0, W0), (W0, CH - W0)] + [(ci * CH, CH) for ci in range(1, NCHUNK)]

    def issue(piece, bi):
        start, n = piece
        g = pltpu.async_copy(
            x_hbm.at[idx_v.at[pl.ds(start, n)]], gth_v[bi].at[pl.ds(0, n)], gsem[bi]
        )
        o = pltpu.async_copy(
            x_hbm.at[pl.ds(stripe + start, n)], own_v[bi].at[pl.ds(0, n)], osem[bi]
        )
        return g, o

    handles = issue(pieces[0], 0)
    for pi, piece in enumerate(pieces):
        bi = pi % 2
        nxt = None
        if pi + 1 < len(pieces):
            nxt = issue(pieces[pi + 1], 1 - bi)
        handles[0].wait()
        handles[1].wait()

        ov, gv = own_v[bi], gth_v[bi]
        off, n = piece

        @plsc.parallel_loop(0, n, step=1, unroll=RUNROLL)
        def _rows(r):
            compute_row(r, off + r, ov, gv)

        handles = nxt

    @plsc.parallel_loop(0, BPW // 16, step=1, unroll=4)
    def _exp(rb):
        out_v[pl.ds(rb * 16, 16)] = jnp.exp(out_v[pl.ds(rb * 16, 16)])

    pltpu.sync_copy(out_v, out_hbm.at[pl.ds(stripe, BPW)])


def kernel(x, idx):
    return _sim_kernel(x, idx).reshape(B, 1)
